# Initial kernel scaffold; baseline (speedup 1.0000x reference)
import jax
import jax.numpy as jnp
from jax.experimental import pallas as pl


def kernel(x, edge_index, p, w_ih, w_hh, b_ih, b_hh, W0, bias):
    raise NotImplementedError("write your pallas kernel here")


# speedup vs baseline: 10.3085x; 10.3085x over previous
import functools

import jax
import jax.numpy as jnp
from jax import lax
from jax.experimental import pallas as pl
from jax.experimental.pallas import tpu as pltpu
from jax.experimental.pallas import tpu_sc as plsc

N = 10000
D = 128
E = 320000
NC, NS = 2, 16
NW = NC * NS
CH = 80
E_PAD = NW * CH * 128
N_PAD = 10240
RPT = N_PAD // NS
DEGW = 16

_HIGH = lax.Precision.HIGHEST


def _topk_gru_body(x_ref, p_ref, wih_ref, whh_ref, bih_ref, bhh_ref,
                   w0_ref, wnew_ref, xt_ref):
    lin = (lax.broadcasted_iota(jnp.int32, (CH, 128), 0) * 128
           + lax.broadcasted_iota(jnp.int32, (CH, 128), 1))
    pv = p_ref[...]
    inv = 1.0 / jnp.sqrt(jnp.sum(pv * pv))
    pb = pv.astype(jnp.bfloat16).astype(jnp.float32)
    x3 = x_ref[...].reshape(CH, 128, D).astype(jnp.bfloat16).astype(jnp.float32)
    s0 = jnp.sum(x3 * pb[None, :, :], axis=2) * inv
    s0 = jnp.where(lin < N, s0, -jnp.inf)

    def step(t, s):
        m = jnp.max(s)
        idx = jnp.min(jnp.where(s == m, lin, jnp.int32(2 ** 30)))
        row = x_ref[pl.ds(idx, 1), :]
        xt_ref[pl.ds(t, 1), :] = row * jnp.tanh(m)
        return jnp.where(lin == idx, -jnp.inf, s)

    lax.fori_loop(0, D, step, s0)

    xt = xt_ref[...].astype(jnp.bfloat16)
    gi = jnp.dot(xt, wih_ref[...].astype(jnp.bfloat16),
                 preferred_element_type=jnp.float32) + bih_ref[...]
    gh = jnp.dot(w0_ref[...].astype(jnp.bfloat16),
                 whh_ref[...].astype(jnp.bfloat16),
                 preferred_element_type=jnp.float32) + bhh_ref[...]
    r = jax.nn.sigmoid(gi[:, :D] + gh[:, :D])
    z = jax.nn.sigmoid(gi[:, D:2 * D] + gh[:, D:2 * D])
    n = jnp.tanh(gi[:, 2 * D:] + r * gh[:, 2 * D:])
    wnew_ref[...] = (1.0 - z) * n + z * w0_ref[...]


def _xw_body(x_ref, wnew_ref, degs_ref, xw_ref, y_ref, disb_ref):
    xw = lax.dot_general(x_ref[...].astype(jnp.bfloat16),
                         wnew_ref[...].astype(jnp.bfloat16),
                         (((1,), (1,)), ((), ())),
                         preferred_element_type=jnp.float32)
    degsum = degs_ref[0, :, 0:1] + degs_ref[1, :, 0:1]
    dis = 1.0 / jnp.sqrt(degsum + 1.0)
    disb = jnp.broadcast_to(dis, (128, D))
    xw_ref[...] = xw
    disb_ref[...] = disb
    y_ref[...] = xw * disb


def _final_body(pa_ref, xw_ref, disb_ref, bias_ref, out_ref):
    acc = pa_ref[0] + pa_ref[1]
    disb = disb_ref[...]
    out_ref[...] = disb * acc + disb * disb * xw_ref[...] + bias_ref[...]


EPW = E_PAD // NW
NCH = EPW // 16


@functools.lru_cache(maxsize=None)
def _sc_kernels():
    mesh = plsc.VectorSubcoreMesh(core_axis_name="c", subcore_axis_name="s",
                                  num_cores=NC, num_subcores=NS)

    @functools.partial(
        pl.kernel,
        out_type=jax.ShapeDtypeStruct((NC, N_PAD, D), jnp.float32),
        mesh=mesh,
        scratch_types=[
            pltpu.VMEM((EPW,), jnp.int32),
            pltpu.VMEM((16, D), jnp.float32),
            pltpu.VMEM((RPT // 10, D), jnp.float32),
            pltpu.VMEM_SHARED((N_PAD, D), jnp.float32),
        ],
    )
    def deg_kernel(cols_hbm, out_hbm, colv, ones_v, zv, acc):
        cid = lax.axis_index("c")
        sid = lax.axis_index("s")
        wid = sid * NC + cid
        pltpu.sync_copy(cols_hbm.at[wid], colv)

        def fill(i, _):
            for l in range(D // 16):
                ones_v[i, pl.ds(l * 16, 16)] = jnp.ones((16,), jnp.float32)
            return 0

        lax.fori_loop(0, 16, fill, 0)

        def fillz(i, _):
            for l in range(D // 16):
                zv[i, pl.ds(l * 16, 16)] = jnp.zeros((16,), jnp.float32)
            return 0

        lax.fori_loop(0, RPT // 10, fillz, 0)
        for t in range(10):
            pltpu.sync_copy(zv, acc.at[pl.ds(sid * RPT + t * (RPT // 10),
                                             RPT // 10)])
        plsc.subcore_barrier()

        def body(j, _):
            iv = colv[pl.ds(j * 16, 16)]
            pltpu.sync_copy(ones_v, acc.at[iv], add=True)
            return 0

        lax.fori_loop(0, NCH, body, 0)
        plsc.subcore_barrier()
        pltpu.sync_copy(acc.at[pl.ds(sid * RPT, RPT)],
                        out_hbm.at[cid, pl.ds(sid * RPT, RPT)])

    @functools.partial(
        pl.kernel,
        out_type=jax.ShapeDtypeStruct((NC, N_PAD, D), jnp.float32),
        mesh=mesh,
        scratch_types=[
            pltpu.VMEM((EPW,), jnp.int32),
            pltpu.VMEM((EPW,), jnp.int32),
            pltpu.VMEM((2, 16, D), jnp.float32),
            pltpu.VMEM((RPT // 10, D), jnp.float32),
            pltpu.VMEM_SHARED((N_PAD, D), jnp.float32),
            pltpu.SemaphoreType.DMA,
            pltpu.SemaphoreType.DMA,
        ],
    )
    def scatter_kernel(y_hbm, rows_hbm, cols_hbm, out_hbm,
                       rowv, colv, buf, zv, acc, sem0, sem1):
        cid = lax.axis_index("c")
        sid = lax.axis_index("s")
        wid = sid * NC + cid
        pltpu.sync_copy(rows_hbm.at[wid], rowv)
        pltpu.sync_copy(cols_hbm.at[wid], colv)

        def fillz(i, _):
            for l in range(D // 16):
                zv[i, pl.ds(l * 16, 16)] = jnp.zeros((16,), jnp.float32)
            return 0

        lax.fori_loop(0, RPT // 10, fillz, 0)
        for t in range(10):
            pltpu.sync_copy(zv, acc.at[pl.ds(sid * RPT + t * (RPT // 10),
                                             RPT // 10)])
        plsc.subcore_barrier()

        sems = (sem0, sem1)
        for b in range(2):
            iv = rowv[pl.ds(b * 16, 16)]
            pltpu.async_copy(y_hbm.at[iv], buf.at[b], sems[b])

        def body(g, _):
            for b in range(2):
                j = g * 2 + b
                iv_r = rowv[pl.ds(j * 16, 16)]
                pltpu.make_async_copy(y_hbm.at[iv_r], buf.at[b], sems[b]).wait()
                iv_c = colv[pl.ds(j * 16, 16)]
                pltpu.sync_copy(buf.at[b], acc.at[iv_c], add=True)
                iv_n = rowv[pl.ds((j + 2) * 16, 16)]
                pltpu.async_copy(y_hbm.at[iv_n], buf.at[b], sems[b])
            return 0

        lax.fori_loop(0, NCH // 2 - 1, body, 0)
        for b in range(2):
            j = NCH - 2 + b
            iv_r = rowv[pl.ds(j * 16, 16)]
            pltpu.make_async_copy(y_hbm.at[iv_r], buf.at[b], sems[b]).wait()
            iv_c = colv[pl.ds(j * 16, 16)]
            pltpu.sync_copy(buf.at[b], acc.at[iv_c], add=True)
        plsc.subcore_barrier()
        pltpu.sync_copy(acc.at[pl.ds(sid * RPT, RPT)],
                        out_hbm.at[cid, pl.ds(sid * RPT, RPT)])

    return deg_kernel, scatter_kernel


def kernel(x, edge_index, p, w_ih, w_hh, b_ih, b_hh, W0, bias):
    x = x.astype(jnp.float32)
    row = edge_index[0].astype(jnp.int32)
    col = edge_index[1].astype(jnp.int32)
    pad = E_PAD - E
    rows3 = jnp.concatenate([row, jnp.zeros((pad,), jnp.int32)]).reshape(NW, EPW)
    cols3 = jnp.concatenate([col, jnp.full((pad,), N, jnp.int32)]).reshape(NW, EPW)
    x_pad = jnp.concatenate([x, jnp.zeros((N_PAD - N, D), jnp.float32)])

    p2 = p.reshape(1, D).astype(jnp.float32)
    bias2 = bias.reshape(1, D).astype(jnp.float32)
    wih_t = w_ih.T.astype(jnp.float32)
    whh_t = w_hh.T.astype(jnp.float32)
    bih2 = b_ih.reshape(1, 3 * D).astype(jnp.float32)
    bhh2 = b_hh.reshape(1, 3 * D).astype(jnp.float32)

    f32 = jnp.float32

    W_new = pl.pallas_call(
        _topk_gru_body,
        out_shape=jax.ShapeDtypeStruct((D, D), f32),
        scratch_shapes=[pltpu.VMEM((D, D), f32)],
    )(x_pad, p2, wih_t, whh_t, bih2, bhh2, W0)

    deg_k, scatter_k = _sc_kernels()
    degs = deg_k(cols3)

    xw, y, disb = pl.pallas_call(
        _xw_body,
        grid=(N_PAD // 128,),
        in_specs=[pl.BlockSpec((128, D), lambda i: (i, 0)),
                  pl.BlockSpec((D, D), lambda i: (0, 0)),
                  pl.BlockSpec((NC, 128, D), lambda i: (0, i, 0))],
        out_specs=[pl.BlockSpec((128, D), lambda i: (i, 0))] * 3,
        out_shape=[jax.ShapeDtypeStruct((N_PAD, D), f32)] * 3,
    )(x_pad, W_new, degs)

    partials = scatter_k(y, rows3, cols3)

    out_full = pl.pallas_call(
        _final_body,
        grid=(N_PAD // 128,),
        in_specs=[pl.BlockSpec((NC, 128, D), lambda i: (0, i, 0)),
                  pl.BlockSpec((128, D), lambda i: (i, 0)),
                  pl.BlockSpec((128, D), lambda i: (i, 0)),
                  pl.BlockSpec((1, D), lambda i: (0, 0))],
        out_specs=pl.BlockSpec((128, D), lambda i: (i, 0)),
        out_shape=jax.ShapeDtypeStruct((N_PAD, D), f32),
    )(partials, xw, disb, bias2)

    return out_full[:N]
